# baseline (speedup 1.0000x reference)
import functools

import jax
import jax.numpy as jnp
from jax import lax
from jax.experimental import pallas as pl
from jax.experimental.pallas import tpu as pltpu
from jax.experimental.pallas import tpu_sc as plsc

_ALPHA = 0.5
_B = 4096
_NCLS = 100000
_D = 128
_NC = 2
_NS = 16
_NW = _NC * _NS
_BPW = _B // _NW
_LANES = 16
_IB = 512


def _seg_mean_body(tf_col_ref, tf_blk_ref, f_ref, centers_any, sm_ref,
                   newc_any, sem):
    i = pl.program_id(0)
    copy = pltpu.make_async_copy(centers_any, newc_any, sem)

    @pl.when(i == 0)
    def _start():
        copy.start()

    a = (tf_col_ref[...] == tf_blk_ref[...]).astype(jnp.float32)
    s = lax.dot_general(a, f_ref[...], (((0,), (0,)), ((), ())),
                        preferred_element_type=jnp.float32)
    cnt = jnp.sum(a, axis=0)
    sm_ref[...] = s / cnt[:, None]

    @pl.when(i == _B // _IB - 1)
    def _drain():
        copy.wait()


def _seg_mean_and_copy(tf_col, tf_row, features, centers):
    return pl.pallas_call(
        _seg_mean_body,
        grid=(_B // _IB,),
        in_specs=[
            pl.BlockSpec((_B, 1), lambda i: (0, 0)),
            pl.BlockSpec((1, _IB), lambda i: (0, i)),
            pl.BlockSpec((_B, _D), lambda i: (0, 0)),
            pl.BlockSpec(memory_space=pltpu.MemorySpace.HBM),
        ],
        out_specs=[
            pl.BlockSpec((_IB, _D), lambda i: (i, 0)),
            pl.BlockSpec(memory_space=pltpu.MemorySpace.HBM),
        ],
        out_shape=[
            jax.ShapeDtypeStruct((_B, _D), jnp.float32),
            jax.ShapeDtypeStruct((_NCLS, _D), jnp.float32),
        ],
        scratch_shapes=[pltpu.SemaphoreType.DMA],
    )(tf_col, tf_row, features, centers)


@functools.lru_cache(maxsize=1)
def _make_sc_update():
    mesh = plsc.VectorSubcoreMesh(core_axis_name="c", subcore_axis_name="s",
                                  num_cores=_NC, num_subcores=_NS)
    return functools.partial(
        pl.kernel,
        out_type=jax.ShapeDtypeStruct((_NW, _LANES), jnp.float32),
        mesh=mesh,
        scratch_types=[
            pltpu.VMEM((_BPW,), jnp.int32),
            pltpu.VMEM((_BPW, _D), jnp.float32),
            pltpu.VMEM((_BPW, _D), jnp.float32),
            pltpu.VMEM((_BPW, _D), jnp.float32),
            pltpu.VMEM((_LANES,), jnp.float32),
            pltpu.SemaphoreType.DMA,
        ],
    )(_sc_update_body)


def _sc_update_body(centers_hbm, targets_hbm, feats_hbm, sm_hbm, newc_ref,
                    loss_out, idx_v, rows_v, feat_v, sm_v, acc_v, sem):
    wid = lax.axis_index("s") * _NC + lax.axis_index("c")
    base = wid * _BPW
    pltpu.sync_copy(targets_hbm.at[pl.ds(base, _BPW)], idx_v)
    gather = pltpu.async_copy(centers_hbm.at[idx_v], rows_v, sem)
    pltpu.sync_copy(feats_hbm.at[pl.ds(base, _BPW)], feat_v)
    pltpu.sync_copy(sm_hbm.at[pl.ds(base, _BPW)], sm_v)
    gather.wait()

    def row_body(r, acc):
        for c in range(_D // _LANES):
            sl = pl.ds(c * _LANES, _LANES)
            g = rows_v[r, sl]
            f = feat_v[r, sl]
            m = sm_v[r, sl]
            d = f - g
            acc = acc + d * d
            sm_v[r, sl] = (1.0 - _ALPHA) * g + _ALPHA * m
        return acc

    acc = lax.fori_loop(0, _BPW, row_body, jnp.zeros((_LANES,), jnp.float32))
    acc_v[...] = acc * (0.5 / _B)
    pltpu.sync_copy(acc_v, loss_out.at[wid])
    pltpu.sync_copy(sm_v, newc_ref.at[idx_v])


def kernel(features, targets, centers):
    tf = targets.astype(jnp.float32)
    sm, newc = _seg_mean_and_copy(tf.reshape(_B, 1), tf.reshape(1, _B),
                                  features, centers)
    newc_ref = jax.new_ref(newc)
    loss_parts = _make_sc_update()(centers, targets, features, sm, newc_ref)
    loss = jnp.sum(loss_parts)
    return loss, newc_ref[...]


# speedup vs baseline: 21.1951x; 21.1951x over previous
import functools

import jax
import jax.numpy as jnp
from jax import lax
from jax.experimental import pallas as pl
from jax.experimental.pallas import tpu as pltpu
from jax.experimental.pallas import tpu_sc as plsc

_ALPHA = 0.5
_B = 4096
_NCLS = 100000
_D = 128
_NC = 2
_NS = 16
_NW = _NC * _NS
_BPW = _B // _NW
_LANES = 16
_IB = 256


_CB = 5000
_NSTEPS = _NCLS // _CB
_NMM = _B // _IB


def _seg_mean_body(tf_col_ref, tf_blk_ref, f_ref, cblk_ref, sm_ref,
                   newc_ref):
    i = pl.program_id(0)
    newc_ref[...] = cblk_ref[...]

    @pl.when(i < _NMM)
    def _mm():
        a = (tf_col_ref[...] == tf_blk_ref[...]).astype(jnp.float32)
        s = lax.dot_general(a, f_ref[...], (((0,), (0,)), ((), ())),
                            preferred_element_type=jnp.float32)
        cnt = jnp.sum(a, axis=0)
        sm_ref[...] = s / cnt[:, None]


def _seg_mean_and_copy(tf_col, tf_row, features, centers):
    mm = lambda i: lax.min(i, _NMM - 1)
    return pl.pallas_call(
        _seg_mean_body,
        grid=(_NSTEPS,),
        in_specs=[
            pl.BlockSpec((_B, 1), lambda i: (0, 0)),
            pl.BlockSpec((1, _IB), lambda i: (0, mm(i))),
            pl.BlockSpec((_B, _D), lambda i: (0, 0)),
            pl.BlockSpec((_CB, _D), lambda i: (i, 0)),
        ],
        out_specs=[
            pl.BlockSpec((_IB, _D), lambda i: (mm(i), 0)),
            pl.BlockSpec((_CB, _D), lambda i: (i, 0)),
        ],
        out_shape=[
            jax.ShapeDtypeStruct((_B, _D), jnp.float32),
            jax.ShapeDtypeStruct((_NCLS, _D), jnp.float32),
        ],
    )(tf_col, tf_row, features, centers)


@functools.lru_cache(maxsize=1)
def _make_sc_update():
    mesh = plsc.VectorSubcoreMesh(core_axis_name="c", subcore_axis_name="s",
                                  num_cores=_NC, num_subcores=_NS)
    return functools.partial(
        pl.kernel,
        out_type=jax.ShapeDtypeStruct((_NW, _LANES), jnp.float32),
        mesh=mesh,
        scratch_types=[
            pltpu.VMEM((_BPW,), jnp.int32),
            pltpu.VMEM((_BPW, _D), jnp.float32),
            pltpu.VMEM((_BPW, _D), jnp.float32),
            pltpu.VMEM((_BPW, _D), jnp.float32),
            pltpu.VMEM((_LANES,), jnp.float32),
            pltpu.SemaphoreType.DMA,
        ],
    )(_sc_update_body)


def _sc_update_body(centers_hbm, targets_hbm, feats_hbm, sm_hbm, newc_ref,
                    loss_out, idx_v, rows_v, feat_v, sm_v, acc_v, sem):
    wid = lax.axis_index("s") * _NC + lax.axis_index("c")
    base = wid * _BPW
    pltpu.sync_copy(targets_hbm.at[pl.ds(base, _BPW)], idx_v)
    gather = pltpu.async_copy(centers_hbm.at[idx_v], rows_v, sem)
    pltpu.sync_copy(feats_hbm.at[pl.ds(base, _BPW)], feat_v)
    pltpu.sync_copy(sm_hbm.at[pl.ds(base, _BPW)], sm_v)
    gather.wait()

    def row_body(r, acc):
        for c in range(_D // _LANES):
            sl = pl.ds(c * _LANES, _LANES)
            g = rows_v[r, sl]
            f = feat_v[r, sl]
            m = sm_v[r, sl]
            d = f - g
            acc = acc + d * d
            sm_v[r, sl] = (1.0 - _ALPHA) * g + _ALPHA * m
        return acc

    acc = lax.fori_loop(0, _BPW, row_body, jnp.zeros((_LANES,), jnp.float32))
    acc_v[...] = acc * (0.5 / _B)
    pltpu.sync_copy(acc_v, loss_out.at[wid])
    pltpu.sync_copy(sm_v, newc_ref.at[idx_v])


def kernel(features, targets, centers):
    tf = targets.astype(jnp.float32)
    sm, newc = _seg_mean_and_copy(tf.reshape(_B, 1), tf.reshape(1, _B),
                                  features, centers)
    newc_ref = jax.new_ref(newc)
    loss_parts = _make_sc_update()(centers, targets, features, sm, newc_ref)
    loss = jnp.sum(loss_parts)
    return loss, newc_ref[...]
